# Initial kernel scaffold; baseline (speedup 1.0000x reference)
import jax
import jax.numpy as jnp
from jax.experimental import pallas as pl


def kernel(logits, noise, k):
    raise NotImplementedError("write your pallas kernel here")


# speedup vs baseline: 3.3568x; 3.3568x over previous
import functools

import jax
import jax.numpy as jnp
from jax import lax
from jax.experimental import pallas as pl
from jax.experimental.pallas import tpu as pltpu
from jax.experimental.pallas import tpu_sc as plsc

B = 128
N = 100000
K = 15
QIDX = 38
W = 2048
NBLK = 49
NPAD = W * NBLK
BIGI = 2**31 - 1

NW = 32
CPW = 64
NG = CPW // 16


def _topk_body(lg_ref, nz_ref, idx_ref, mn_ref, mx_ref, cv, ci, mns, mxs):
    b = pl.program_id(0)

    @pl.when(b == 0)
    def _init():
        cv[...] = jnp.full((B, 128), -jnp.inf, jnp.float32)
        ci[...] = jnp.full((B, 128), 2**31 - 1, jnp.int32)
        mns[0, 0] = jnp.float32(jnp.inf)
        mxs[0, 0] = jnp.float32(-jnp.inf)

    x = lg_ref[...]
    u = jnp.clip(nz_ref[...], 1e-8, 1.0 - 1e-8)
    z = -jnp.log(-jnp.log(u))
    ii = lax.broadcasted_iota(jnp.int32, (B, W), 1)
    valid = (b * W + ii) < N
    p = jnp.where(valid, x + z, -jnp.inf)

    mns[0, 0] = jnp.minimum(mns[0, 0], jnp.min(jnp.where(valid, x, jnp.inf)))
    mxs[0, 0] = jnp.maximum(mxs[0, 0], jnp.max(jnp.where(valid, x, -jnp.inf)))

    for s in range(K):
        m = jnp.max(p, axis=1, keepdims=True)
        lid = jnp.min(jnp.where(p == m, ii, BIGI), axis=1, keepdims=True)
        cv[:, K + s:K + s + 1] = m
        ci[:, K + s:K + s + 1] = b * W + lid
        p = jnp.where(ii == lid, -jnp.inf, p)

    v = cv[...]
    iid = ci[...]
    selv, seli = [], []
    for s in range(K):
        m = jnp.max(v, axis=1, keepdims=True)
        sid = jnp.min(jnp.where(v == m, iid, BIGI), axis=1, keepdims=True)
        selv.append(m)
        seli.append(sid)
        v = jnp.where((v == m) & (iid == sid), -jnp.inf, v)
    cv[...] = jnp.concatenate(
        selv + [jnp.full((B, 128 - K), -jnp.inf, jnp.float32)], axis=1)
    ci[...] = jnp.concatenate(
        seli + [jnp.full((B, 128 - K), 2**31 - 1, jnp.int32)], axis=1)

    @pl.when(b == NBLK - 1)
    def _fin():
        idx_ref[...] = ci[...]
        mn_ref[0, 0] = mns[0, 0]
        mx_ref[0, 0] = mxs[0, 0]


def _topk_call(lp, nz):
    return pl.pallas_call(
        _topk_body,
        grid=(NBLK,),
        in_specs=[
            pl.BlockSpec((B, W), lambda b: (0, b)),
            pl.BlockSpec((B, W), lambda b: (0, b)),
        ],
        out_specs=[
            pl.BlockSpec((B, 128), lambda b: (0, 0)),
            pl.BlockSpec(memory_space=pltpu.SMEM),
            pl.BlockSpec(memory_space=pltpu.SMEM),
        ],
        out_shape=[
            jax.ShapeDtypeStruct((B, 128), jnp.int32),
            jax.ShapeDtypeStruct((1, 1), jnp.float32),
            jax.ShapeDtypeStruct((1, 1), jnp.float32),
        ],
        scratch_shapes=[
            pltpu.VMEM((B, 128), jnp.float32),
            pltpu.VMEM((B, 128), jnp.int32),
            pltpu.SMEM((1, 1), jnp.float32),
            pltpu.SMEM((1, 1), jnp.float32),
        ],
        compiler_params=pltpu.CompilerParams(
            dimension_semantics=("arbitrary",)),
    )(lp, nz)


def _sc_body(flat_hbm, idxp_hbm, mn_hbm, mx_hbm, out_hbm,
             cols_v, gidx_v, gdat_v, xidx_v, xdat_v, out_v, mn_v, mx_v, sem):
    c = lax.axis_index("c")
    s = lax.axis_index("s")
    wid = s * 2 + c
    base = wid * CPW

    pltpu.sync_copy(idxp_hbm.at[wid], cols_v)
    pltpu.sync_copy(mn_hbm, mn_v)
    pltpu.sync_copy(mx_hbm, mx_v)
    mn = mn_v[...]
    mx = mx_v[...]
    lanes = lax.iota(jnp.int32, 16)

    for g in range(NG):
        cvec = cols_v[pl.ds(g * 16, 16)]
        ivec = jnp.minimum(lax.div(base + g * 16 + lanes, jnp.int32(K)), B - 1)
        xidx_v[pl.ds(g * 16, 16)] = cvec + ivec * N
        pltpu.make_async_copy(
            flat_hbm.at[xidx_v.at[pl.ds(g * 16, 16)]],
            xdat_v.at[pl.ds(g * 16, 16)], sem).start()

        def build(r, carry):
            gidx_v[pl.ds((g * B + r) * 16, 16)] = cvec + r * N
            return carry

        lax.fori_loop(0, B, build, 0)

        def fire(r, carry):
            pltpu.make_async_copy(
                flat_hbm.at[gidx_v.at[pl.ds((g * B + r) * 16, 16)]],
                gdat_v.at[pl.ds((g * B + r) * 16, 16)], sem).start()
            return carry

        lax.fori_loop(0, B, fire, 0)

    def drainall(t, carry):
        pltpu.make_async_copy(
            flat_hbm.at[gidx_v.at[pl.ds(t * 16, 16)]],
            gdat_v.at[pl.ds(t * 16, 16)], sem).wait()
        return carry

    lax.fori_loop(0, NG * B, drainall, 0)
    for g in range(NG):
        pltpu.make_async_copy(
            flat_hbm.at[xidx_v.at[pl.ds(g * 16, 16)]],
            xdat_v.at[pl.ds(g * 16, 16)], sem).wait()

    for g in range(NG):
        ai = (xdat_v[pl.ds(g * 16, 16)] - mn) / mx

        def count(r, cnt):
            ar = (gdat_v[pl.ds((g * B + r) * 16, 16)] - mn) / mx
            return cnt + jnp.where(ar <= ai, 1, 0).astype(jnp.int32)

        cnt = lax.fori_loop(0, B, count, jnp.zeros((16,), jnp.int32))
        val = jnp.where(cnt <= QIDX, jnp.zeros((16,), jnp.float32), ai)
        out_v[pl.ds(g * 16, 16)] = val

    pltpu.sync_copy(out_v, out_hbm.at[wid])


def _sc_call(flat_logits, idx_pad, mn16, mx16):
    mesh = plsc.VectorSubcoreMesh(core_axis_name="c", subcore_axis_name="s")
    fn = functools.partial(
        pl.kernel,
        out_type=jax.ShapeDtypeStruct((NW, CPW), jnp.float32),
        mesh=mesh,
        scratch_types=[
            pltpu.VMEM((CPW,), jnp.int32),
            pltpu.VMEM((NG * B * 16,), jnp.int32),
            pltpu.VMEM((NG * B * 16,), jnp.float32),
            pltpu.VMEM((NG * 16,), jnp.int32),
            pltpu.VMEM((NG * 16,), jnp.float32),
            pltpu.VMEM((CPW,), jnp.float32),
            pltpu.VMEM((16,), jnp.float32),
            pltpu.VMEM((16,), jnp.float32),
            pltpu.SemaphoreType.DMA,
        ],
    )(_sc_body)
    return fn(flat_logits, idx_pad, mn16, mx16)


def kernel(logits, noise, k):
    del k
    lp = jnp.pad(logits, ((0, 0), (0, NPAD - N)))
    nz = jnp.pad(noise, ((0, 0), (0, NPAD - N)), constant_values=0.5)
    idx128, mn, mx = _topk_call(lp, nz)
    idxk = idx128[:, :K]

    flat_idx = idxk.reshape(-1)
    idx_pad = jnp.pad(flat_idx, (0, NW * CPW - B * K)).reshape(NW, CPW)
    mn16 = jnp.broadcast_to(mn.reshape(()), (16,))
    mx16 = jnp.broadcast_to(mx.reshape(()), (16,))
    out2d = _sc_call(logits.reshape(-1), idx_pad, mn16, mx16)
    att = out2d.reshape(-1)[:B * K].reshape(B, K)
    return att, idxk


# speedup vs baseline: 4.0923x; 1.2191x over previous
import functools

import jax
import jax.numpy as jnp
from jax import lax
from jax.experimental import pallas as pl
from jax.experimental.pallas import tpu as pltpu
from jax.experimental.pallas import tpu_sc as plsc

B = 128
N = 100000
K = 15
QIDX = 38
W = 2048
NBLK = 49
NPAD = W * NBLK
BIGI = 2**31 - 1

NW = 32
CPW = 64
NG = CPW // 16


def _topk_body(lg_ref, nz_ref, idx_ref, mn_ref, mx_ref, lt_ref,
               cv, ci, mns, mxs):
    b = pl.program_id(0)

    @pl.when(b == 0)
    def _init():
        cv[...] = jnp.full((B, 128), -jnp.inf, jnp.float32)
        ci[...] = jnp.full((B, 128), 2**31 - 1, jnp.int32)
        mns[0, 0] = jnp.float32(jnp.inf)
        mxs[0, 0] = jnp.float32(-jnp.inf)

    x = lg_ref[...]
    lt_ref[...] = x.T
    u = jnp.clip(nz_ref[...], 1e-8, 1.0 - 1e-8)
    z = -jnp.log(-jnp.log(u))
    ii = lax.broadcasted_iota(jnp.int32, (B, W), 1)
    valid = (b * W + ii) < N
    p = jnp.where(valid, x + z, -jnp.inf)

    mns[0, 0] = jnp.minimum(mns[0, 0], jnp.min(jnp.where(valid, x, jnp.inf)))
    mxs[0, 0] = jnp.maximum(mxs[0, 0], jnp.max(jnp.where(valid, x, -jnp.inf)))

    for s in range(K):
        m = jnp.max(p, axis=1, keepdims=True)
        lid = jnp.min(jnp.where(p == m, ii, BIGI), axis=1, keepdims=True)
        cv[:, K + s:K + s + 1] = m
        ci[:, K + s:K + s + 1] = b * W + lid
        p = jnp.where(ii == lid, -jnp.inf, p)

    v = cv[...]
    iid = ci[...]
    selv, seli = [], []
    for s in range(K):
        m = jnp.max(v, axis=1, keepdims=True)
        sid = jnp.min(jnp.where(v == m, iid, BIGI), axis=1, keepdims=True)
        selv.append(m)
        seli.append(sid)
        v = jnp.where((v == m) & (iid == sid), -jnp.inf, v)
    cv[...] = jnp.concatenate(
        selv + [jnp.full((B, 128 - K), -jnp.inf, jnp.float32)], axis=1)
    ci[...] = jnp.concatenate(
        seli + [jnp.full((B, 128 - K), 2**31 - 1, jnp.int32)], axis=1)

    @pl.when(b == NBLK - 1)
    def _fin():
        idx_ref[...] = ci[...]
        mn_ref[0, 0] = mns[0, 0]
        mx_ref[0, 0] = mxs[0, 0]


def _topk_call(lg, nz):
    return pl.pallas_call(
        _topk_body,
        grid=(NBLK,),
        in_specs=[
            pl.BlockSpec((B, W), lambda b: (0, b)),
            pl.BlockSpec((B, W), lambda b: (0, b)),
        ],
        out_specs=[
            pl.BlockSpec((B, 128), lambda b: (0, 0)),
            pl.BlockSpec(memory_space=pltpu.SMEM),
            pl.BlockSpec(memory_space=pltpu.SMEM),
            pl.BlockSpec((W, B), lambda b: (b, 0)),
        ],
        out_shape=[
            jax.ShapeDtypeStruct((B, 128), jnp.int32),
            jax.ShapeDtypeStruct((1, 1), jnp.float32),
            jax.ShapeDtypeStruct((1, 1), jnp.float32),
            jax.ShapeDtypeStruct((NPAD, B), jnp.float32),
        ],
        scratch_shapes=[
            pltpu.VMEM((B, 128), jnp.float32),
            pltpu.VMEM((B, 128), jnp.int32),
            pltpu.SMEM((1, 1), jnp.float32),
            pltpu.SMEM((1, 1), jnp.float32),
        ],
        compiler_params=pltpu.CompilerParams(
            dimension_semantics=("arbitrary",)),
    )(lg, nz)


def _sc_body(tab_hbm, idxp_hbm, out_hbm, cols_v, gdat_v, sem):
    c = lax.axis_index("c")
    s = lax.axis_index("s")
    wid = s * 2 + c

    pltpu.sync_copy(idxp_hbm.at[wid], cols_v)
    cp = pltpu.make_async_copy(tab_hbm.at[cols_v], gdat_v, sem)
    cp.start()
    cp.wait()
    pltpu.sync_copy(gdat_v, out_hbm.at[pl.ds(wid * CPW, CPW)])


def _sc_call(table, idx_pad):
    mesh = plsc.VectorSubcoreMesh(core_axis_name="c", subcore_axis_name="s")
    fn = functools.partial(
        pl.kernel,
        out_type=jax.ShapeDtypeStruct((NW * CPW, B), jnp.float32),
        mesh=mesh,
        scratch_types=[
            pltpu.VMEM((CPW,), jnp.int32),
            pltpu.VMEM((CPW, B), jnp.float32),
            pltpu.SemaphoreType.DMA,
        ],
    )(_sc_body)
    return fn(table, idx_pad)


TPAD = NW * CPW


def _att_body(g_ref, mn_ref, mx_ref, out_ref):
    x = g_ref[...]
    mn = mn_ref[0, 0]
    mx = mx_ref[0, 0]
    a = (x - mn) / mx
    rows = lax.broadcasted_iota(jnp.int32, (TPAD, B), 0)
    cols = lax.broadcasted_iota(jnp.int32, (TPAD, B), 1)
    imap = jnp.minimum(rows // K, B - 1)
    sel = (cols == imap).astype(jnp.float32)
    ai = jnp.sum(a * sel, axis=1, keepdims=True)
    cnt = jnp.sum((a <= ai).astype(jnp.int32), axis=1, keepdims=True)
    out_ref[...] = jnp.where(cnt <= QIDX, 0.0, ai)


def _att_call(g, mn, mx):
    return pl.pallas_call(
        _att_body,
        in_specs=[
            pl.BlockSpec((TPAD, B), lambda: (0, 0)),
            pl.BlockSpec(memory_space=pltpu.SMEM),
            pl.BlockSpec(memory_space=pltpu.SMEM),
        ],
        out_specs=pl.BlockSpec((TPAD, 1), lambda: (0, 0)),
        out_shape=jax.ShapeDtypeStruct((TPAD, 1), jnp.float32),
    )(g, mn, mx)


def kernel(logits, noise, k):
    del k
    idx128, mn, mx, table = _topk_call(logits, noise)
    idxk = idx128[:, :K]

    flat_idx = idxk.reshape(-1)
    idx_pad = jnp.pad(flat_idx, (0, TPAD - B * K)).reshape(NW, CPW)
    g = _sc_call(table, idx_pad)
    att2 = _att_call(g, mn, mx)
    att = att2.reshape(-1)[:B * K].reshape(B, K)
    return att, idxk


# speedup vs baseline: 6.7408x; 1.6472x over previous
import functools

import jax
import jax.numpy as jnp
from jax import lax
from jax.experimental import pallas as pl
from jax.experimental.pallas import tpu as pltpu
from jax.experimental.pallas import tpu_sc as plsc

B = 128
N = 100000
K = 15
QIDX = 38
W = 2048
NBLK = 49
NPAD = W * NBLK
BIGI = 2**31 - 1

NW = 32
CPW = 64
NG = CPW // 16


CAND = NBLK * 128
BIGF = 1e9


def _topk_body(lg_ref, nz_ref, idx_ref, mn_ref, mx_ref, lt_ref,
               cand_v, cand_i, mns, mxs):
    b = pl.program_id(0)

    @pl.when(b == 0)
    def _init():
        cand_v[...] = jnp.full((B, CAND), -jnp.inf, jnp.float32)
        cand_i[...] = jnp.full((B, CAND), BIGF, jnp.float32)
        mns[0, 0] = jnp.float32(jnp.inf)
        mxs[0, 0] = jnp.float32(-jnp.inf)

    x = lg_ref[...]
    lt_ref[...] = x.T
    u = jnp.clip(nz_ref[...], 1e-8, 1.0 - 1e-8)
    z = -jnp.log(-jnp.log(u))
    iif = lax.broadcasted_iota(jnp.int32, (B, W), 1).astype(jnp.float32)
    last = b == NBLK - 1

    @pl.when(jnp.logical_not(last))
    def _mm_full():
        mns[0, 0] = jnp.minimum(mns[0, 0], jnp.min(x))
        mxs[0, 0] = jnp.maximum(mxs[0, 0], jnp.max(x))

    @pl.when(last)
    def _mm_masked():
        ii = lax.broadcasted_iota(jnp.int32, (B, W), 1)
        valid = (b * W + ii) < N
        mns[0, 0] = jnp.minimum(
            mns[0, 0], jnp.min(jnp.where(valid, x, jnp.inf)))
        mxs[0, 0] = jnp.maximum(
            mxs[0, 0], jnp.max(jnp.where(valid, x, -jnp.inf)))

    lim = jnp.where(last, jnp.float32(N - (NBLK - 1) * W), jnp.float32(W))
    p = jnp.where(iif < lim, x + z, -jnp.inf)

    bwf = (b * W).astype(jnp.float32)
    selv, seli = [], []
    for s in range(K):
        m = jnp.max(p, axis=1, keepdims=True)
        lid = jnp.min(jnp.where(p == m, iif, BIGF), axis=1, keepdims=True)
        selv.append(m)
        seli.append(lid + bwf)
        p = jnp.where(iif == lid, -jnp.inf, p)
    bv = jnp.concatenate(
        selv + [jnp.full((B, 128 - K), -jnp.inf, jnp.float32)], axis=1)
    bi = jnp.concatenate(
        seli + [jnp.full((B, 128 - K), BIGF, jnp.float32)], axis=1)
    cand_v[:, pl.ds(b * 128, 128)] = bv
    cand_i[:, pl.ds(b * 128, 128)] = bi

    @pl.when(last)
    def _fin():
        v = cand_v[...]
        iid = cand_i[...]
        sel2 = []
        for s in range(K):
            m = jnp.max(v, axis=1, keepdims=True)
            sid = jnp.min(jnp.where(v == m, iid, BIGF), axis=1, keepdims=True)
            sel2.append(sid)
            v = jnp.where((v == m) & (iid == sid), -jnp.inf, v)
        ids = jnp.concatenate(
            sel2 + [jnp.zeros((B, 1), jnp.float32)], axis=1)
        idx_ref[...] = ids.astype(jnp.int32)
        mn_ref[0, 0] = mns[0, 0]
        mx_ref[0, 0] = mxs[0, 0]


def _topk_call(lg, nz):
    return pl.pallas_call(
        _topk_body,
        grid=(NBLK,),
        in_specs=[
            pl.BlockSpec((B, W), lambda b: (0, b)),
            pl.BlockSpec((B, W), lambda b: (0, b)),
        ],
        out_specs=[
            pl.BlockSpec((B, 16), lambda b: (0, 0)),
            pl.BlockSpec(memory_space=pltpu.SMEM),
            pl.BlockSpec(memory_space=pltpu.SMEM),
            pl.BlockSpec((W, B), lambda b: (b, 0)),
        ],
        out_shape=[
            jax.ShapeDtypeStruct((B, 16), jnp.int32),
            jax.ShapeDtypeStruct((1, 1), jnp.float32),
            jax.ShapeDtypeStruct((1, 1), jnp.float32),
            jax.ShapeDtypeStruct((NPAD, B), jnp.float32),
        ],
        scratch_shapes=[
            pltpu.VMEM((B, CAND), jnp.float32),
            pltpu.VMEM((B, CAND), jnp.float32),
            pltpu.SMEM((1, 1), jnp.float32),
            pltpu.SMEM((1, 1), jnp.float32),
        ],
        compiler_params=pltpu.CompilerParams(
            dimension_semantics=("arbitrary",)),
    )(lg, nz)


def _sc_body(tab_hbm, idxp_hbm, out_hbm, cols_v, gdat_v, sem):
    c = lax.axis_index("c")
    s = lax.axis_index("s")
    wid = s * 2 + c

    pltpu.sync_copy(idxp_hbm.at[wid], cols_v)
    cp = pltpu.make_async_copy(tab_hbm.at[cols_v], gdat_v, sem)
    cp.start()
    cp.wait()
    pltpu.sync_copy(gdat_v, out_hbm.at[pl.ds(wid * CPW, CPW)])


def _sc_call(table, idx_pad):
    mesh = plsc.VectorSubcoreMesh(core_axis_name="c", subcore_axis_name="s")
    fn = functools.partial(
        pl.kernel,
        out_type=jax.ShapeDtypeStruct((NW * CPW, B), jnp.float32),
        mesh=mesh,
        scratch_types=[
            pltpu.VMEM((CPW,), jnp.int32),
            pltpu.VMEM((CPW, B), jnp.float32),
            pltpu.SemaphoreType.DMA,
        ],
    )(_sc_body)
    return fn(table, idx_pad)


TPAD = NW * CPW


def _att_body(g_ref, mn_ref, mx_ref, out_ref):
    x = g_ref[...]
    mn = mn_ref[0, 0]
    mx = mx_ref[0, 0]
    a = (x - mn) / mx
    rows = lax.broadcasted_iota(jnp.int32, (TPAD, B), 0)
    cols = lax.broadcasted_iota(jnp.int32, (TPAD, B), 1)
    imap = jnp.minimum(rows // K, B - 1)
    sel = (cols == imap).astype(jnp.float32)
    ai = jnp.sum(a * sel, axis=1, keepdims=True)
    cnt = jnp.sum((a <= ai).astype(jnp.int32), axis=1, keepdims=True)
    out_ref[...] = jnp.where(cnt <= QIDX, 0.0, ai)


def _att_call(g, mn, mx):
    return pl.pallas_call(
        _att_body,
        in_specs=[
            pl.BlockSpec((TPAD, B), lambda: (0, 0)),
            pl.BlockSpec(memory_space=pltpu.SMEM),
            pl.BlockSpec(memory_space=pltpu.SMEM),
        ],
        out_specs=pl.BlockSpec((TPAD, 1), lambda: (0, 0)),
        out_shape=jax.ShapeDtypeStruct((TPAD, 1), jnp.float32),
    )(g, mn, mx)


def kernel(logits, noise, k):
    del k
    idx128, mn, mx, table = _topk_call(logits, noise)
    idxk = idx128[:, :K]

    flat_idx = idxk.reshape(-1)
    idx_pad = jnp.pad(flat_idx, (0, TPAD - B * K)).reshape(NW, CPW)
    g = _sc_call(table, idx_pad)
    att2 = _att_call(g, mn, mx)
    att = att2.reshape(-1)[:B * K].reshape(B, K)
    return att, idxk
